# baseline (speedup 1.0000x reference)
import jax
import jax.numpy as jnp
from jax.experimental import pallas as pl
from jax.experimental.pallas import tpu as pltpu

_TM = 2048
_NBUF = 8
_NSTEP = 48


def _body(out_ref, buf_ref, sems):
    buf_ref[...] = jnp.full(buf_ref.shape, 1.0, jnp.float32)

    def loop_body(j, carry):
        @pl.when(j >= _NBUF)
        def _():
            pltpu.make_async_copy(
                buf_ref, out_ref.at[:, pl.ds((j - _NBUF) * _TM, _TM)],
                sems.at[j % _NBUF]).wait()
        pltpu.make_async_copy(
            buf_ref, out_ref.at[:, pl.ds(j * _TM, _TM)],
            sems.at[j % _NBUF]).start()
        return carry

    jax.lax.fori_loop(0, _NSTEP, loop_body, 0)

    def drain(j, carry):
        pltpu.make_async_copy(
            buf_ref, out_ref.at[:, pl.ds(j * _TM, _TM)],
            sems.at[j % _NBUF]).wait()
        return carry

    jax.lax.fori_loop(_NSTEP - _NBUF, _NSTEP, drain, 0)


def kernel(inputs, mem, epoch, roi_labels):
    B, D = inputs.shape
    M = mem.shape[0]
    return pl.pallas_call(
        _body,
        out_specs=pl.BlockSpec(memory_space=pl.ANY),
        out_shape=jax.ShapeDtypeStruct((B, M), jnp.float32),
        scratch_shapes=[
            pltpu.VMEM((B, _TM), jnp.float32),
            pltpu.SemaphoreType.DMA((_NBUF,)),
        ],
    )()


# speedup vs baseline: 3.2603x; 3.2603x over previous
import jax
import jax.numpy as jnp
from jax.experimental import pallas as pl
from jax.experimental.pallas import tpu as pltpu

_TM = 2048


def _body(x_ref, lab_ref, mem_ref, out_ref):
    mask = lab_ref[...] > 0
    x = jnp.where(mask, x_ref[...], 0.0).astype(jnp.bfloat16)
    m = mem_ref[...].astype(jnp.bfloat16)
    out_ref[...] = jax.lax.dot_general(
        m, x, (((1,), (1,)), ((), ())), preferred_element_type=jnp.float32
    )


def kernel(inputs, mem, epoch, roi_labels):
    B, D = inputs.shape
    M = mem.shape[0]
    labels = roi_labels.reshape(B, 1)
    out_t = pl.pallas_call(
        _body,
        grid=(pl.cdiv(M, _TM),),
        in_specs=[
            pl.BlockSpec((B, D), lambda j: (0, 0)),
            pl.BlockSpec((B, 1), lambda j: (0, 0)),
            pl.BlockSpec((_TM, D), lambda j: (j, 0)),
        ],
        out_specs=pl.BlockSpec((_TM, B), lambda j: (j, 0)),
        out_shape=jax.ShapeDtypeStruct((M, B), jnp.float32),
        compiler_params=pltpu.CompilerParams(
            dimension_semantics=("parallel",),
        ),
    )(inputs, labels, mem)
    return out_t.T


# speedup vs baseline: 3.2949x; 1.0106x over previous
import jax
import jax.numpy as jnp
from jax.experimental import pallas as pl
from jax.experimental.pallas import tpu as pltpu

_TM = 2000


def _body(x_ref, lab_ref, mem_ref, out_ref):
    mask = jnp.reshape(lab_ref[...], (lab_ref.shape[1], 1)) > 0
    x = jnp.where(mask, x_ref[...], 0.0).astype(jnp.bfloat16)
    m = mem_ref[...].astype(jnp.bfloat16)
    out_ref[...] = jax.lax.dot_general(
        m, x, (((1,), (1,)), ((), ())), preferred_element_type=jnp.float32
    )


def kernel(inputs, mem, epoch, roi_labels):
    B, D = inputs.shape
    M = mem.shape[0]
    out_t = pl.pallas_call(
        _body,
        grid=(pl.cdiv(M, _TM),),
        in_specs=[
            pl.BlockSpec((B, D), lambda j: (0, 0)),
            pl.BlockSpec((1, B), lambda j: (0, 0)),
            pl.BlockSpec((_TM, D), lambda j: (j, 0)),
        ],
        out_specs=pl.BlockSpec((_TM, B), lambda j: (j, 0)),
        out_shape=jax.ShapeDtypeStruct((M, B), jnp.float32),
        compiler_params=pltpu.CompilerParams(
            dimension_semantics=("parallel",),
        ),
    )(inputs, roi_labels, mem)
    return out_t.T


# speedup vs baseline: 3.3869x; 1.0279x over previous
import jax
import jax.numpy as jnp
from jax.experimental import pallas as pl
from jax.experimental.pallas import tpu as pltpu

_TM = 4000


def _body(x_ref, lab_ref, mem_ref, out_ref):
    mask = jnp.reshape(lab_ref[...], (lab_ref.shape[1], 1)) > 0
    x = jnp.where(mask, x_ref[...], 0.0).astype(jnp.bfloat16)
    m = mem_ref[...].astype(jnp.bfloat16)
    out_ref[...] = jax.lax.dot_general(
        m, x, (((1,), (1,)), ((), ())), preferred_element_type=jnp.float32
    )


def kernel(inputs, mem, epoch, roi_labels):
    B, D = inputs.shape
    M = mem.shape[0]
    out_t = pl.pallas_call(
        _body,
        grid=(pl.cdiv(M, _TM),),
        in_specs=[
            pl.BlockSpec((B, D), lambda j: (0, 0)),
            pl.BlockSpec((1, B), lambda j: (0, 0)),
            pl.BlockSpec((_TM, D), lambda j: (j, 0)),
        ],
        out_specs=pl.BlockSpec((_TM, B), lambda j: (j, 0)),
        out_shape=jax.ShapeDtypeStruct((M, B), jnp.float32),
        compiler_params=pltpu.CompilerParams(
            dimension_semantics=("parallel",),
        ),
    )(inputs, roi_labels, mem)
    return out_t.T


# speedup vs baseline: 3.3966x; 1.0029x over previous
import jax
import jax.numpy as jnp
from jax.experimental import pallas as pl
from jax.experimental.pallas import tpu as pltpu

_TM = 5000


def _body(x_ref, lab_ref, mem_ref, out_ref):
    mask = jnp.reshape(lab_ref[...], (lab_ref.shape[1], 1)) > 0
    x = jnp.where(mask, x_ref[...], 0.0).astype(jnp.bfloat16)
    m = mem_ref[...].astype(jnp.bfloat16)
    out_ref[...] = jax.lax.dot_general(
        m, x, (((1,), (1,)), ((), ())), preferred_element_type=jnp.float32
    )


def kernel(inputs, mem, epoch, roi_labels):
    B, D = inputs.shape
    M = mem.shape[0]
    out_t = pl.pallas_call(
        _body,
        grid=(pl.cdiv(M, _TM),),
        in_specs=[
            pl.BlockSpec((B, D), lambda j: (0, 0)),
            pl.BlockSpec((1, B), lambda j: (0, 0)),
            pl.BlockSpec((_TM, D), lambda j: (j, 0)),
        ],
        out_specs=pl.BlockSpec((_TM, B), lambda j: (j, 0)),
        out_shape=jax.ShapeDtypeStruct((M, B), jnp.float32),
        compiler_params=pltpu.CompilerParams(
            dimension_semantics=("parallel",),
        ),
    )(inputs, roi_labels, mem)
    return out_t.T
